# Initial kernel scaffold; baseline (speedup 1.0000x reference)
import jax
import jax.numpy as jnp
from jax.experimental import pallas as pl


def kernel(input, index, _):
    raise NotImplementedError("write your pallas kernel here")


# speedup vs baseline: 2.4256x; 2.4256x over previous
import functools

import jax
import jax.numpy as jnp
from jax import lax
from jax.experimental import pallas as pl
from jax.experimental.pallas import tpu as pltpu
from jax.experimental.pallas import tpu_sc as plsc

NUM_INPUTS = 160000
NUM_FEATURES = 256
NUM_OUTPUTS = 10000

NC = 2
NS = 16
FH = NUM_FEATURES // NC
ROWS_PER_SUB = NUM_INPUTS // NS
CHUNK = 80
NCHUNK = ROWS_PER_SUB // CHUNK
OUT_PER_SUB = 640
ACC_ROWS = NS * OUT_PER_SUB
OCHUNK = 80
FULL_OCHUNKS = OUT_PER_SUB // OCHUNK
LAST_OCHUNKS = (NUM_OUTPUTS - 15 * OUT_PER_SUB) // OCHUNK

_mesh = plsc.VectorSubcoreMesh(core_axis_name="c", subcore_axis_name="s")


@functools.partial(
    pl.kernel,
    out_type=jax.ShapeDtypeStruct((NUM_OUTPUTS, NUM_FEATURES), jnp.float32),
    mesh=_mesh,
    scratch_types=[
        pltpu.VMEM((CHUNK,), jnp.int32),
        pltpu.VMEM((CHUNK, FH), jnp.float32),
        pltpu.VMEM((OCHUNK, FH), jnp.float32),
        pltpu.VMEM_SHARED((ACC_ROWS, FH), jnp.float32),
    ],
)
def _sc_segment_sum(inp_hbm, idx_hbm, z_hbm, out_hbm, idx_v, rows_v, obuf, acc):
    c = lax.axis_index("c")
    s = lax.axis_index("s")
    col0 = c * FH
    obase = s * OUT_PER_SUB

    pltpu.sync_copy(z_hbm, obuf)
    for k in range(FULL_OCHUNKS):
        pltpu.sync_copy(obuf, acc.at[pl.ds(obase + k * OCHUNK, OCHUNK)])
    plsc.subcore_barrier()

    rbase = s * ROWS_PER_SUB

    def body(i, carry):
        base = rbase + i * CHUNK
        pltpu.sync_copy(idx_hbm.at[pl.ds(base, CHUNK)], idx_v)
        pltpu.sync_copy(inp_hbm.at[pl.ds(base, CHUNK), pl.ds(col0, FH)], rows_v)
        pltpu.sync_copy(rows_v, acc.at[idx_v], add=True)
        return carry

    lax.fori_loop(0, NCHUNK, body, 0)
    plsc.subcore_barrier()

    nk = jnp.where(s == NS - 1, LAST_OCHUNKS, FULL_OCHUNKS)

    def out_body(k, carry):
        r0 = obase + k * OCHUNK
        pltpu.sync_copy(acc.at[pl.ds(r0, OCHUNK)], obuf)
        pltpu.sync_copy(obuf, out_hbm.at[pl.ds(r0, OCHUNK), pl.ds(col0, FH)])
        return carry

    lax.fori_loop(0, nk, out_body, 0)


def kernel(input, index, _):
    z = jnp.zeros((OCHUNK, FH), jnp.float32)
    out = _sc_segment_sum(input, index, z)
    return (input, index, out)


# speedup vs baseline: 4.0988x; 1.6898x over previous
import functools

import jax
import jax.numpy as jnp
from jax import lax
from jax.experimental import pallas as pl
from jax.experimental.pallas import tpu as pltpu
from jax.experimental.pallas import tpu_sc as plsc

NUM_INPUTS = 160000
NUM_FEATURES = 256
NUM_OUTPUTS = 10000

NC = 2
NS = 16
FH = NUM_FEATURES // NC
ROWS_PER_SUB = NUM_INPUTS // NS
CHUNK = 40
NCHUNK = ROWS_PER_SUB // CHUNK
OUT_PER_SUB = 640
ACC_ROWS = NS * OUT_PER_SUB
OCHUNK = 80
FULL_OCHUNKS = OUT_PER_SUB // OCHUNK
LAST_OCHUNKS = (NUM_OUTPUTS - 15 * OUT_PER_SUB) // OCHUNK
NBUF = 5
NSTEPS = NCHUNK // NBUF

_mesh = plsc.VectorSubcoreMesh(core_axis_name="c", subcore_axis_name="s")


@functools.partial(
    pl.kernel,
    out_type=jax.ShapeDtypeStruct((NUM_OUTPUTS, NUM_FEATURES), jnp.float32),
    mesh=_mesh,
    scratch_types=[
        [pltpu.VMEM((CHUNK,), jnp.int32) for _ in range(NBUF)],
        [pltpu.VMEM((CHUNK, FH), jnp.float32) for _ in range(NBUF)],
        [pltpu.SemaphoreType.DMA for _ in range(NBUF)],
        [pltpu.SemaphoreType.DMA for _ in range(NBUF)],
        pltpu.VMEM((OCHUNK, FH), jnp.float32),
        pltpu.VMEM_SHARED((ACC_ROWS, FH), jnp.float32),
    ],
)
def _sc_segment_sum(inp_hbm, idx_hbm, z_hbm, out_hbm,
                    idxs, rows, isems, rsems, obuf, acc):
    c = lax.axis_index("c")
    s = lax.axis_index("s")
    col0 = c * FH
    obase = s * OUT_PER_SUB
    rbase = s * ROWS_PER_SUB

    def issue_loads(i, b):
        base = rbase + i * CHUNK
        pltpu.async_copy(idx_hbm.at[pl.ds(base, CHUNK)], idxs[b], isems[b])
        pltpu.async_copy(
            inp_hbm.at[pl.ds(base, CHUNK), pl.ds(col0, FH)], rows[b], rsems[b])

    def wait_loads(b):
        pltpu.make_async_copy(
            idx_hbm.at[pl.ds(0, CHUNK)], idxs[b], isems[b]).wait()
        pltpu.make_async_copy(
            inp_hbm.at[pl.ds(0, CHUNK), pl.ds(0, FH)], rows[b], rsems[b]).wait()

    for b in range(NBUF):
        issue_loads(b, b)
    pltpu.sync_copy(z_hbm, obuf)
    for k in range(FULL_OCHUNKS):
        pltpu.sync_copy(obuf, acc.at[pl.ds(obase + k * OCHUNK, OCHUNK)])
    plsc.subcore_barrier()

    def step_body(step, carry):
        for b in range(NBUF):
            i = step * NBUF + b
            wait_loads(b)
            pltpu.sync_copy(rows[b], acc.at[idxs[b]], add=True)

            @pl.when(step < NSTEPS - 1)
            def _():
                issue_loads(i + NBUF, b)
        return carry

    lax.fori_loop(0, NSTEPS, step_body, 0)
    plsc.subcore_barrier()

    nk = jnp.where(s == NS - 1, LAST_OCHUNKS, FULL_OCHUNKS)

    def out_body(k, carry):
        r0 = obase + k * OCHUNK
        pltpu.sync_copy(acc.at[pl.ds(r0, OCHUNK)], obuf)
        pltpu.sync_copy(obuf, out_hbm.at[pl.ds(r0, OCHUNK), pl.ds(col0, FH)])
        return carry

    lax.fori_loop(0, nk, out_body, 0)


def kernel(input, index, _):
    z = jnp.zeros((OCHUNK, FH), jnp.float32)
    out = _sc_segment_sum(input, index, z)
    return (input, index, out)
